# Initial kernel scaffold; baseline (speedup 1.0000x reference)
import jax
import jax.numpy as jnp
from jax.experimental import pallas as pl


def kernel(x, edge_index, batch, W1, b1, W2, b2):
    raise NotImplementedError("write your pallas kernel here")


# speedup vs baseline: 11.2185x; 11.2185x over previous
import functools

import jax
import jax.numpy as jnp
from jax import lax
from jax.experimental import pallas as pl
from jax.experimental.pallas import tpu as pltpu
from jax.experimental.pallas import tpu_sc as plsc

_N = 10000
_E = 320000
_D = 128
_G = 64

_NC, _NS = 2, 16
_NW = _NC * _NS
_CH = 128
_EPW = ((_E + _NW * _CH - 1) // (_NW * _CH)) * _CH
_E_PAD = _EPW * _NW
_NCH = _EPW // _CH
_N_PAD = 10240
_RPT = _N_PAD // _NS
_RB = 1000

_mesh = plsc.VectorSubcoreMesh(
    core_axis_name="c", subcore_axis_name="s", num_cores=_NC, num_subcores=_NS
)


@functools.partial(
    pl.kernel,
    out_type=jax.ShapeDtypeStruct((_NC * _N_PAD,), jnp.float32),
    mesh=_mesh,
    scratch_types=[
        pltpu.VMEM((_CH,), jnp.int32),
        pltpu.VMEM((_CH,), jnp.float32),
        pltpu.VMEM((_RPT,), jnp.float32),
        pltpu.VMEM_SHARED((_N_PAD,), jnp.float32),
    ],
)
def _deg_kernel(dst_hbm, out_hbm, dst_v, ones_v, zero_v, accum):
    c = lax.axis_index("c")
    s = lax.axis_index("s")
    for j in range(_CH // 16):
        ones_v[pl.ds(j * 16, 16)] = jnp.ones((16,), jnp.float32)
    for j in range(_RPT // 16):
        zero_v[pl.ds(j * 16, 16)] = jnp.zeros((16,), jnp.float32)
    pltpu.sync_copy(zero_v, accum.at[pl.ds(s * _RPT, _RPT)])
    plsc.subcore_barrier()
    base = (c * _NS + s) * _EPW

    def body(k, carry):
        pltpu.sync_copy(dst_hbm.at[pl.ds(base + k * _CH, _CH)], dst_v)
        pltpu.sync_copy(ones_v, accum.at[dst_v], add=True)
        return carry

    lax.fori_loop(0, _NCH, body, 0)
    plsc.subcore_barrier()
    pltpu.sync_copy(
        accum.at[pl.ds(s * _RPT, _RPT)],
        out_hbm.at[pl.ds(c * _N_PAD + s * _RPT, _RPT)],
    )


@functools.partial(
    pl.kernel,
    out_type=jax.ShapeDtypeStruct((_NC, _N_PAD, _D), jnp.float32),
    mesh=_mesh,
    scratch_types=[
        pltpu.VMEM((_CH,), jnp.int32),
        pltpu.VMEM((_CH,), jnp.int32),
        pltpu.VMEM((_CH, _D), jnp.float32),
        pltpu.VMEM_SHARED((_N_PAD, _D), jnp.float32),
        pltpu.SemaphoreType.DMA,
    ],
)
def _agg_kernel(src_hbm, dst_hbm, h_hbm, out_hbm, src_v, dst_v, rows_v, accum, sem):
    c = lax.axis_index("c")
    s = lax.axis_index("s")

    def zbody(i, carry):
        for j in range(_D // 16):
            rows_v[i, pl.ds(j * 16, 16)] = jnp.zeros((16,), jnp.float32)
        return carry

    lax.fori_loop(0, _CH, zbody, 0)
    for r in range(_RPT // _CH):
        pltpu.sync_copy(rows_v, accum.at[pl.ds(s * _RPT + r * _CH, _CH)])
    plsc.subcore_barrier()
    base = (c * _NS + s) * _EPW

    def body(k, carry):
        off = base + k * _CH
        pltpu.sync_copy(src_hbm.at[pl.ds(off, _CH)], src_v)
        pltpu.sync_copy(dst_hbm.at[pl.ds(off, _CH)], dst_v)
        pltpu.async_copy(h_hbm.at[src_v], rows_v, sem).wait()
        pltpu.sync_copy(rows_v, accum.at[dst_v], add=True)
        return carry

    lax.fori_loop(0, _NCH, body, 0)
    plsc.subcore_barrier()
    for r in range(_RPT // _CH):
        pltpu.sync_copy(
            accum.at[pl.ds(s * _RPT + r * _CH, _CH)],
            out_hbm.at[c, pl.ds(s * _RPT + r * _CH, _CH)],
        )


def _prep_body(x_ref, w_ref, d0_ref, d1_ref, hp_ref, dinv_ref):
    deg = d0_ref[...] + d1_ref[...] + 1.0
    dinv = lax.rsqrt(deg)
    h = jnp.dot(x_ref[...], w_ref[...], preferred_element_type=jnp.float32)
    hp_ref[...] = h * dinv
    dinv_ref[...] = dinv


def _mid_body(s0_ref, s1_ref, hp_ref, dinv_ref, b_ref, w_ref, out_ref):
    S = s0_ref[0] + s1_ref[0]
    dinv = dinv_ref[...]
    z = jnp.maximum(dinv * (S + hp_ref[...]) + b_ref[...], 0.0)
    out_ref[...] = jnp.dot(z, w_ref[...], preferred_element_type=jnp.float32) * dinv


def _final_body(s0_ref, s1_ref, hp_ref, dinv_ref, b_ref, bat_ref, out_ref, acc_s, acc_c):
    i = pl.program_id(0)
    S = s0_ref[0] + s1_ref[0]
    dinv = dinv_ref[...]
    z = jnp.maximum(dinv * (S + hp_ref[...]) + b_ref[...], 0.0)
    gid = lax.broadcasted_iota(jnp.int32, (_RB, _G), 1)
    m = (bat_ref[...] == gid).astype(jnp.float32)
    dn = (((0,), (0,)), ((), ()))

    @pl.when(i == 0)
    def _():
        acc_s[...] = jnp.zeros_like(acc_s)
        acc_c[...] = jnp.zeros_like(acc_c)

    acc_s[...] += lax.dot_general(m, z, dn, preferred_element_type=jnp.float32)
    acc_c[...] += lax.dot_general(
        m, jnp.ones((_RB, _D), jnp.float32), dn, preferred_element_type=jnp.float32
    )

    @pl.when(i == pl.num_programs(0) - 1)
    def _():
        out_ref[...] = acc_s[...] / jnp.maximum(acc_c[...], 1.0)


def _row_spec():
    return pl.BlockSpec((_RB, _D), lambda i: (i, 0))


def _col1_spec(dtype_unused=None):
    return pl.BlockSpec((_RB, 1), lambda i: (i, 0))


def _full_spec(shape):
    return pl.BlockSpec(shape, lambda i: tuple(0 for _ in shape))


_prep_call = pl.pallas_call(
    _prep_body,
    grid=(_N // _RB,),
    in_specs=[_row_spec(), _full_spec((_D, _D)), _col1_spec(), _col1_spec()],
    out_specs=[_row_spec(), _col1_spec()],
    out_shape=[
        jax.ShapeDtypeStruct((_N, _D), jnp.float32),
        jax.ShapeDtypeStruct((_N, 1), jnp.float32),
    ],
)

_part_spec0 = pl.BlockSpec((1, _RB, _D), lambda i: (0, i, 0))
_part_spec1 = pl.BlockSpec((1, _RB, _D), lambda i: (1, i, 0))

_mid_call = pl.pallas_call(
    _mid_body,
    grid=(_N // _RB,),
    in_specs=[
        _part_spec0, _part_spec1, _row_spec(), _col1_spec(),
        _full_spec((1, _D)), _full_spec((_D, _D)),
    ],
    out_specs=_row_spec(),
    out_shape=jax.ShapeDtypeStruct((_N, _D), jnp.float32),
)

_final_call = pl.pallas_call(
    _final_body,
    grid=(_N // _RB,),
    in_specs=[
        _part_spec0, _part_spec1, _row_spec(), _col1_spec(),
        _full_spec((1, _D)), _col1_spec(),
    ],
    out_specs=_full_spec((_G, _D)),
    out_shape=jax.ShapeDtypeStruct((_G, _D), jnp.float32),
    scratch_shapes=[
        pltpu.VMEM((_G, _D), jnp.float32),
        pltpu.VMEM((_G, _D), jnp.float32),
    ],
)


def kernel(x, edge_index, batch, W1, b1, W2, b2):
    src = edge_index[0]
    dst = edge_index[1]
    pad = _E_PAD - _E
    src_p = jnp.concatenate([src, jnp.zeros((pad,), jnp.int32)])
    dst_p = jnp.concatenate([dst, jnp.full((pad,), _N, jnp.int32)])

    degp = _deg_kernel(dst_p)
    d0 = degp[:_N].reshape(_N, 1)
    d1 = degp[_N_PAD:_N_PAD + _N].reshape(_N, 1)

    hp1, dinv = _prep_call(x, W1, d0, d1)
    S1 = _agg_kernel(src_p, dst_p, hp1)
    hp2 = _mid_call(S1, S1, hp1, dinv, b1.reshape(1, _D), W2)
    S2 = _agg_kernel(src_p, dst_p, hp2)
    out = _final_call(S2, S2, hp2, dinv, b2.reshape(1, _D), batch.reshape(_N, 1))
    return out
